# baseline (speedup 1.0000x reference)
import jax
import jax.numpy as jnp
from jax import lax
from jax.experimental import pallas as pl
from jax.experimental.pallas import tpu as pltpu

_B = 256
_HE = 1024
_HD = 1024
_N = 32640
_T = 384


def _bdot(a, b):
    return jnp.dot(a, b, preferred_element_type=jnp.float32)


def _p_kernel(prev_ref, w1t_ref, b_ref, out_ref):
    out_ref[...] = _bdot(prev_ref[...], w1t_ref[...]) + b_ref[...]


def _scores_kernel(starts_ref, ends_ref, enc_ref, w2t_ref, phi_ref, plo_ref,
                   plo2_ref, v_ref, out_ref, mx_ref):
    t = pl.program_id(0)

    @pl.when(t == 0)
    def _init():
        mx_ref[...] = jnp.full((1, _B), -jnp.inf, jnp.float32)

    rows = t * _T + lax.broadcasted_iota(jnp.int32, (_T, 1), 0)
    in_seg = (rows >= starts_ref[...]) & (rows < ends_ref[...])
    oh = in_seg.astype(jnp.bfloat16)
    pre = _bdot(enc_ref[...], w2t_ref[...])
    pre = pre + (_bdot(oh, phi_ref[...]) + _bdot(oh, plo_ref[...])
                 + _bdot(oh, plo2_ref[...]))
    energy = jnp.tanh(pre).astype(jnp.bfloat16)
    s = _bdot(energy, v_ref[...])
    out_ref[...] = s
    vals = jnp.where(in_seg, s, -jnp.inf)
    mx_ref[...] = jnp.maximum(mx_ref[...], jnp.max(vals, axis=0, keepdims=True))


def _expsum_kernel(starts_ref, ends_ref, s_ref, mx_ref, out_ref):
    t = pl.program_id(0)

    @pl.when(t == 0)
    def _init():
        out_ref[...] = jnp.zeros((1, _B), jnp.float32)

    rows = t * _T + lax.broadcasted_iota(jnp.int32, (_T, 1), 0)
    in_seg = (rows >= starts_ref[...]) & (rows < ends_ref[...])
    mrow = jnp.sum(jnp.where(in_seg, mx_ref[...], 0.0), axis=1, keepdims=True)
    ex = jnp.exp(s_ref[...] - mrow)
    out_ref[...] += jnp.sum(jnp.where(in_seg, ex, 0.0), axis=0, keepdims=True)


def _norm_kernel(starts_ref, ends_ref, s_ref, mx_ref, den_ref, out_ref):
    t = pl.program_id(0)
    rows = t * _T + lax.broadcasted_iota(jnp.int32, (_T, 1), 0)
    in_seg = (rows >= starts_ref[...]) & (rows < ends_ref[...])
    mrow = jnp.sum(jnp.where(in_seg, mx_ref[...], 0.0), axis=1, keepdims=True)
    drow = jnp.sum(jnp.where(in_seg, den_ref[...], 0.0), axis=1, keepdims=True)
    out_ref[...] = jnp.exp(s_ref[...] - mrow) / drow


def kernel(prev_hidden_states, encoder_output, tree_sizes, W, b, v):
    w1t = W[:, :_HD].T.astype(jnp.bfloat16)
    w2t = W[:, _HD:].T.astype(jnp.bfloat16)
    enc16 = encoder_output.astype(jnp.bfloat16)
    prev16 = prev_hidden_states.astype(jnp.bfloat16)
    csum = jnp.cumsum(tree_sizes.astype(jnp.int32))
    starts = jnp.concatenate(
        [jnp.zeros((1,), jnp.int32), csum[:-1]]).reshape(1, _B)
    ends = csum.reshape(1, _B)
    b2 = b.reshape(1, _HE)
    v16 = v.reshape(_HE, 1).astype(jnp.bfloat16)

    p_tab = pl.pallas_call(
        _p_kernel,
        out_shape=jax.ShapeDtypeStruct((_B, _HE), jnp.float32),
    )(prev16, w1t, b2)
    p_hi = p_tab.astype(jnp.bfloat16)
    r1 = p_tab - p_hi.astype(jnp.float32)
    p_lo = r1.astype(jnp.bfloat16)
    p_lo2 = (r1 - p_lo.astype(jnp.float32)).astype(jnp.bfloat16)

    grid = _N // _T
    scores, segmax = pl.pallas_call(
        _scores_kernel,
        grid=(grid,),
        in_specs=[
            pl.BlockSpec((1, _B), lambda t: (0, 0)),
            pl.BlockSpec((1, _B), lambda t: (0, 0)),
            pl.BlockSpec((_T, _HE), lambda t: (t, 0)),
            pl.BlockSpec((_HE, _HE), lambda t: (0, 0)),
            pl.BlockSpec((_B, _HE), lambda t: (0, 0)),
            pl.BlockSpec((_B, _HE), lambda t: (0, 0)),
            pl.BlockSpec((_B, _HE), lambda t: (0, 0)),
            pl.BlockSpec((_HE, 1), lambda t: (0, 0)),
        ],
        out_specs=[
            pl.BlockSpec((_T, 1), lambda t: (t, 0)),
            pl.BlockSpec((1, _B), lambda t: (0, 0)),
        ],
        out_shape=[
            jax.ShapeDtypeStruct((_N, 1), jnp.float32),
            jax.ShapeDtypeStruct((1, _B), jnp.float32),
        ],
    )(starts, ends, enc16, w2t, p_hi, p_lo, p_lo2, v16)

    densum = pl.pallas_call(
        _expsum_kernel,
        grid=(grid,),
        in_specs=[
            pl.BlockSpec((1, _B), lambda t: (0, 0)),
            pl.BlockSpec((1, _B), lambda t: (0, 0)),
            pl.BlockSpec((_T, 1), lambda t: (t, 0)),
            pl.BlockSpec((1, _B), lambda t: (0, 0)),
        ],
        out_specs=pl.BlockSpec((1, _B), lambda t: (0, 0)),
        out_shape=jax.ShapeDtypeStruct((1, _B), jnp.float32),
    )(starts, ends, scores, segmax)

    att = pl.pallas_call(
        _norm_kernel,
        grid=(grid,),
        in_specs=[
            pl.BlockSpec((1, _B), lambda t: (0, 0)),
            pl.BlockSpec((1, _B), lambda t: (0, 0)),
            pl.BlockSpec((_T, 1), lambda t: (t, 0)),
            pl.BlockSpec((1, _B), lambda t: (0, 0)),
            pl.BlockSpec((1, _B), lambda t: (0, 0)),
        ],
        out_specs=pl.BlockSpec((_T, 1), lambda t: (t, 0)),
        out_shape=jax.ShapeDtypeStruct((_N, 1), jnp.float32),
    )(starts, ends, scores, segmax, densum)

    return att


# speedup vs baseline: 1.2689x; 1.2689x over previous
import jax
import jax.numpy as jnp
from jax import lax
from jax.experimental import pallas as pl
from jax.experimental.pallas import tpu as pltpu

_B = 256
_HE = 1024
_HD = 1024
_N = 32640
_T = 384


def _bdot(a, b):
    return jnp.dot(a, b, preferred_element_type=jnp.float32)


def _p_kernel(prev_ref, w1t_ref, b_ref, hi_ref, lo_ref):
    p = _bdot(prev_ref[...].astype(jnp.bfloat16), w1t_ref[...]) + b_ref[...]
    hi = p.astype(jnp.bfloat16)
    hi_ref[...] = hi
    lo_ref[...] = (p - hi.astype(jnp.float32)).astype(jnp.bfloat16)


def _scores_kernel(starts_ref, ends_ref, enc_ref, w2t_ref, phi_ref, plo_ref,
                   v_ref, out_ref, mx_ref):
    t = pl.program_id(0)

    @pl.when(t == 0)
    def _init():
        mx_ref[...] = jnp.full((1, _B), -jnp.inf, jnp.float32)

    rows = t * _T + lax.broadcasted_iota(jnp.int32, (_T, 1), 0)
    in_seg = (rows >= starts_ref[...]) & (rows < ends_ref[...])
    oh = in_seg.astype(jnp.bfloat16)
    pre = _bdot(enc_ref[...].astype(jnp.bfloat16), w2t_ref[...])
    pre = pre + (_bdot(oh, phi_ref[...]) + _bdot(oh, plo_ref[...]))
    energy = jnp.tanh(pre).astype(jnp.bfloat16)
    s = _bdot(energy, v_ref[...])
    out_ref[...] = s
    vals = jnp.where(in_seg, s, -jnp.inf)
    mx_ref[...] = jnp.maximum(mx_ref[...], jnp.max(vals, axis=0, keepdims=True))


def _expsum_kernel(starts_ref, ends_ref, s_ref, mx_ref, out_ref):
    t = pl.program_id(0)

    @pl.when(t == 0)
    def _init():
        out_ref[...] = jnp.zeros((1, _B), jnp.float32)

    rows = t * _T + lax.broadcasted_iota(jnp.int32, (_T, 1), 0)
    in_seg = (rows >= starts_ref[...]) & (rows < ends_ref[...])
    mrow = jnp.sum(jnp.where(in_seg, mx_ref[...], 0.0), axis=1, keepdims=True)
    ex = jnp.exp(s_ref[...] - mrow)
    out_ref[...] += jnp.sum(jnp.where(in_seg, ex, 0.0), axis=0, keepdims=True)


def _norm_kernel(starts_ref, ends_ref, s_ref, mx_ref, den_ref, out_ref):
    t = pl.program_id(0)
    rows = t * _T + lax.broadcasted_iota(jnp.int32, (_T, 1), 0)
    in_seg = (rows >= starts_ref[...]) & (rows < ends_ref[...])
    mrow = jnp.sum(jnp.where(in_seg, mx_ref[...], 0.0), axis=1, keepdims=True)
    drow = jnp.sum(jnp.where(in_seg, den_ref[...], 0.0), axis=1, keepdims=True)
    out_ref[...] = jnp.exp(s_ref[...] - mrow) / drow


def kernel(prev_hidden_states, encoder_output, tree_sizes, W, b, v):
    w1t = W[:, :_HD].T.astype(jnp.bfloat16)
    w2t = W[:, _HD:].T.astype(jnp.bfloat16)
    csum = jnp.cumsum(tree_sizes.astype(jnp.int32))
    starts = jnp.concatenate(
        [jnp.zeros((1,), jnp.int32), csum[:-1]]).reshape(1, _B)
    ends = csum.reshape(1, _B)
    b2 = b.reshape(1, _HE)
    v16 = v.reshape(_HE, 1).astype(jnp.bfloat16)

    p_hi, p_lo = pl.pallas_call(
        _p_kernel,
        out_shape=[
            jax.ShapeDtypeStruct((_B, _HE), jnp.bfloat16),
            jax.ShapeDtypeStruct((_B, _HE), jnp.bfloat16),
        ],
    )(prev_hidden_states, w1t, b2)

    grid = _N // _T
    scores, segmax = pl.pallas_call(
        _scores_kernel,
        grid=(grid,),
        in_specs=[
            pl.BlockSpec((1, _B), lambda t: (0, 0)),
            pl.BlockSpec((1, _B), lambda t: (0, 0)),
            pl.BlockSpec((_T, _HE), lambda t: (t, 0)),
            pl.BlockSpec((_HE, _HE), lambda t: (0, 0)),
            pl.BlockSpec((_B, _HE), lambda t: (0, 0)),
            pl.BlockSpec((_B, _HE), lambda t: (0, 0)),
            pl.BlockSpec((_HE, 1), lambda t: (0, 0)),
        ],
        out_specs=[
            pl.BlockSpec((_T, 1), lambda t: (t, 0)),
            pl.BlockSpec((1, _B), lambda t: (0, 0)),
        ],
        out_shape=[
            jax.ShapeDtypeStruct((_N, 1), jnp.float32),
            jax.ShapeDtypeStruct((1, _B), jnp.float32),
        ],
    )(starts, ends, encoder_output, w2t, p_hi, p_lo, v16)

    densum = pl.pallas_call(
        _expsum_kernel,
        grid=(grid,),
        in_specs=[
            pl.BlockSpec((1, _B), lambda t: (0, 0)),
            pl.BlockSpec((1, _B), lambda t: (0, 0)),
            pl.BlockSpec((_T, 1), lambda t: (t, 0)),
            pl.BlockSpec((1, _B), lambda t: (0, 0)),
        ],
        out_specs=pl.BlockSpec((1, _B), lambda t: (0, 0)),
        out_shape=jax.ShapeDtypeStruct((1, _B), jnp.float32),
    )(starts, ends, scores, segmax)

    att = pl.pallas_call(
        _norm_kernel,
        grid=(grid,),
        in_specs=[
            pl.BlockSpec((1, _B), lambda t: (0, 0)),
            pl.BlockSpec((1, _B), lambda t: (0, 0)),
            pl.BlockSpec((_T, 1), lambda t: (t, 0)),
            pl.BlockSpec((1, _B), lambda t: (0, 0)),
            pl.BlockSpec((1, _B), lambda t: (0, 0)),
        ],
        out_specs=pl.BlockSpec((_T, 1), lambda t: (t, 0)),
        out_shape=jax.ShapeDtypeStruct((_N, 1), jnp.float32),
    )(starts, ends, scores, segmax, densum)

    return att
